# Initial kernel scaffold; baseline (speedup 1.0000x reference)
import jax
import jax.numpy as jnp
from jax.experimental import pallas as pl


def kernel(x, edge_index, edge_weight, W1, b1, g1, be1, W2, b2, g2, be2, W3, b3, g3, be3, Wh, bh, Wa, ba, W2o, b2o):
    raise NotImplementedError("write your pallas kernel here")


# speedup vs baseline: 7.5190x; 7.5190x over previous
import functools
import jax
import jax.numpy as jnp
from jax import lax
from jax.experimental import pallas as pl
from jax.experimental.pallas import tpu as pltpu
from jax.experimental.pallas import tpu_sc as plsc

N = 10000
H = 128
NC = 2
NS = 16
NW = NC * NS
B = 128
N_ACC = 10240
ROWS_PER_SUB = N_ACC // NS
EPS = 1e-5


def _cdiv(a, b):
    return (a + b - 1) // b


def _sc_agg_body(chunks, hs_hbm, src_hbm, dst_hbm, ew_hbm, out_hbm,
                 acc_sh, src_v, dst_v, ew_v, rows_v, sem):
    cid = lax.axis_index("c")
    sid = lax.axis_index("s")

    z16 = jnp.zeros((16,), jnp.float32)

    def zero_row(i, _):
        for c in range(H // 16):
            rows_v[i, pl.ds(c * 16, 16)] = z16
        return 0

    lax.fori_loop(0, B, zero_row, 0)
    row0 = sid * ROWS_PER_SUB
    for k in range(ROWS_PER_SUB // B):
        pltpu.sync_copy(rows_v, acc_sh.at[pl.ds(row0 + k * B, B)])
    plsc.subcore_barrier()

    base = (cid * NS + sid) * (chunks * B)

    def chunk_body(k, _):
        off = base + k * B
        pltpu.sync_copy(src_hbm.at[pl.ds(off, B)], src_v)
        pltpu.sync_copy(dst_hbm.at[pl.ds(off, B)], dst_v)
        pltpu.sync_copy(ew_hbm.at[pl.ds(off, B)], ew_v)
        pltpu.async_copy(hs_hbm.at[src_v], rows_v, sem).wait()

        def scale_grp(g, _):
            ew16 = ew_v[pl.ds(g * 16, 16)]
            for r in range(16):
                w = ew16[r]
                row = g * 16 + r
                for c in range(H // 16):
                    sl = pl.ds(c * 16, 16)
                    rows_v[row, sl] = rows_v[row, sl] * w
            return 0

        lax.fori_loop(0, B // 16, scale_grp, 0)
        pltpu.sync_copy(rows_v, acc_sh.at[dst_v], add=True)
        return 0

    lax.fori_loop(0, chunks, chunk_body, 0)
    plsc.subcore_barrier()

    pltpu.sync_copy(acc_sh.at[pl.ds(row0, ROWS_PER_SUB)],
                    out_hbm.at[cid, pl.ds(row0, ROWS_PER_SUB)])


def _make_sc_agg(chunks):
    mesh = plsc.VectorSubcoreMesh(core_axis_name="c", subcore_axis_name="s")
    return pl.kernel(
        functools.partial(_sc_agg_body, chunks),
        out_type=jax.ShapeDtypeStruct((NC, N_ACC, H), jnp.float32),
        mesh=mesh,
        scratch_types=[
            pltpu.VMEM_SHARED((N_ACC, H), jnp.float32),
            pltpu.VMEM((B,), jnp.int32),
            pltpu.VMEM((B,), jnp.int32),
            pltpu.VMEM((B,), jnp.float32),
            pltpu.VMEM((B, H), jnp.float32),
            pltpu.SemaphoreType.DMA,
        ],
    )


def _sc_deg_body(chunks, dst_hbm, ew_hbm, out_hbm, acc_sh, dst_v, ew_v, zz_v):
    cid = lax.axis_index("c")
    sid = lax.axis_index("s")

    z16 = jnp.zeros((16,), jnp.float32)
    for c in range(B // 16):
        zz_v[pl.ds(c * 16, 16)] = z16
    row0 = sid * ROWS_PER_SUB
    for k in range(ROWS_PER_SUB // B):
        pltpu.sync_copy(zz_v, acc_sh.at[pl.ds(row0 + k * B, B)])
    plsc.subcore_barrier()

    base = (cid * NS + sid) * (chunks * B)

    def chunk_body(k, _):
        off = base + k * B
        pltpu.sync_copy(dst_hbm.at[pl.ds(off, B)], dst_v)
        pltpu.sync_copy(ew_hbm.at[pl.ds(off, B)], ew_v)
        pltpu.sync_copy(ew_v, acc_sh.at[dst_v], add=True)
        return 0

    lax.fori_loop(0, chunks, chunk_body, 0)
    plsc.subcore_barrier()
    pltpu.sync_copy(acc_sh.at[pl.ds(row0, ROWS_PER_SUB)],
                    out_hbm.at[cid, pl.ds(row0, ROWS_PER_SUB)])


def _make_sc_deg(chunks):
    mesh = plsc.VectorSubcoreMesh(core_axis_name="c", subcore_axis_name="s")
    return pl.kernel(
        functools.partial(_sc_deg_body, chunks),
        out_type=jax.ShapeDtypeStruct((NC, N_ACC), jnp.float32),
        mesh=mesh,
        scratch_types=[
            pltpu.VMEM_SHARED((N_ACC,), jnp.float32),
            pltpu.VMEM((B,), jnp.int32),
            pltpu.VMEM((B,), jnp.float32),
            pltpu.VMEM((B,), jnp.float32),
        ],
    )


def _tc_first_body(x_ref, w_ref, dinv_ref, out_ref):
    h = lax.dot_general(x_ref[...], w_ref[...], (((1,), (1,)), ((), ())),
                        preferred_element_type=jnp.float32)
    out_ref[...] = h * dinv_ref[...]


def _tc_mid_body(p_ref, hs_ref, dinv_ref, b_ref, g_ref, be_ref, w_ref, out_ref):
    agg = p_ref[0, :N, :] + p_ref[1, :N, :] + hs_ref[...]
    pre = agg * dinv_ref[...] + b_ref[...]
    mean = jnp.mean(pre, axis=0, keepdims=True)
    var = jnp.mean((pre - mean) ** 2, axis=0, keepdims=True)
    h = jnp.maximum((pre - mean) * lax.rsqrt(var + EPS) * g_ref[...] + be_ref[...], 0.0)
    hn = lax.dot_general(h, w_ref[...], (((1,), (1,)), ((), ())),
                         preferred_element_type=jnp.float32)
    out_ref[...] = hn * dinv_ref[...]


def _tc_last_body(p_ref, hs_ref, dinv_ref, b_ref, g_ref, be_ref,
                  wh_ref, bh_ref, wa_ref, w2o_ref, out_ref):
    agg = p_ref[0, :N, :] + p_ref[1, :N, :] + hs_ref[...]
    pre = agg * dinv_ref[...] + b_ref[...]
    mean = jnp.mean(pre, axis=0, keepdims=True)
    var = jnp.mean((pre - mean) ** 2, axis=0, keepdims=True)
    h = jnp.maximum((pre - mean) * lax.rsqrt(var + EPS) * g_ref[...] + be_ref[...], 0.0)
    h8 = h[0:8, :]
    hh = jnp.maximum(
        lax.dot_general(h8, wh_ref[...], (((1,), (1,)), ((), ())),
                        preferred_element_type=jnp.float32) + bh_ref[...], 0.0)
    r0 = jnp.sum(hh[0:1, :] * wa_ref[...], axis=1, keepdims=True)
    r1 = jnp.sum(hh[1:2, :] * w2o_ref[...], axis=1, keepdims=True)
    out_ref[...] = jnp.concatenate([r0, r1], axis=0)


def _tc_call(body, out_shape, *args):
    return pl.pallas_call(body, out_shape=out_shape)(*args)


def kernel(x, edge_index, edge_weight, W1, b1, g1, be1, W2, b2, g2, be2,
           W3, b3, g3, be3, Wh, bh, Wa, ba, W2o, b2o):
    E = edge_index.shape[1]
    chunks = _cdiv(E, NW * B)
    e_pad = chunks * NW * B

    src = edge_index[0]
    dst = edge_index[1]
    pad = e_pad - E
    if pad:
        src = jnp.concatenate([src, jnp.zeros((pad,), src.dtype)])
        dst = jnp.concatenate([dst, jnp.full((pad,), N, dst.dtype)])
        ew = jnp.concatenate([edge_weight, jnp.zeros((pad,), edge_weight.dtype)])
    else:
        ew = edge_weight

    degp = _make_sc_deg(chunks)(dst, ew)
    deg = degp[0, :N] + degp[1, :N] + 1.0
    dinv = deg ** -0.5
    dinv2 = dinv[:, None]

    b1r = b1[None, :]
    g1r = g1[None, :]
    be1r = be1[None, :]

    sc_agg = _make_sc_agg(chunks)

    hs = _tc_call(_tc_first_body, jax.ShapeDtypeStruct((N, H), jnp.float32),
                  x, W1, dinv2)
    p = sc_agg(hs, src, dst, ew)
    hs = _tc_call(_tc_mid_body, jax.ShapeDtypeStruct((N, H), jnp.float32),
                  p, hs, dinv2, b1r, g1r, be1r, W2)
    p = sc_agg(hs, src, dst, ew)
    hs = _tc_call(_tc_mid_body, jax.ShapeDtypeStruct((N, H), jnp.float32),
                  p, hs, dinv2, b2[None, :], g2[None, :], be2[None, :], W3)
    p = sc_agg(hs, src, dst, ew)
    out2 = _tc_call(_tc_last_body, jax.ShapeDtypeStruct((2, 1), jnp.float32),
                    p, hs, dinv2, b3[None, :], g3[None, :], be3[None, :],
                    Wh, bh[None, :], Wa, W2o)
    return out2[:, 0] + jnp.concatenate([ba, b2o])
